# baseline (speedup 1.0000x reference)
import functools

import jax
import jax.numpy as jnp
from jax import lax
from jax.experimental import pallas as pl
from jax.experimental.pallas import tpu as pltpu

_MARGIN = 0.02
_EPS = 1e-8


def _copy_for(w_hbm, stg_ref, sem, block, k, slot):
    return pltpu.make_async_copy(
        w_hbm.at[pl.ds(k * block, block), :], stg_ref.at[slot],
        sem.at[slot])


def _stream_layer(w_hbm, cache_ref, stg_ref, sem, block, margin, eps, carry):
    n, d = w_hbm.shape
    nb = n // block
    tile_shape = (block, block)
    tri = (lax.broadcasted_iota(jnp.int32, tile_shape, 1) >
           lax.broadcasted_iota(jnp.int32, tile_shape, 0))

    def body_k(k, c1):
        slot = lax.rem(k, 2)

        @pl.when(k + 1 < nb)
        def _prefetch():
            _copy_for(w_hbm, stg_ref, sem, block, k + 1,
                      lax.rem(k + 1, 2)).start()

        _copy_for(w_hbm, stg_ref, sem, block, k, slot).wait()
        f = stg_ref[slot]
        inv_k = 1.0 / jnp.maximum(jnp.sqrt(jnp.sum(f * f, axis=1)), eps)
        bn = (f * inv_k[:, None]).astype(jnp.float8_e4m3fn)
        cache_ref[pl.ds(pl.multiple_of(k * block, block), block), :] = bn

        sim_d = lax.dot_general(
            bn, bn, (((1,), (1,)), ((), ())),
            preferred_element_type=jnp.float32,
        )

        def reduce_tile(sim_prev, diag_if_zero, c2):
            s, cnt = c2
            pos = jnp.logical_and(sim_prev > margin,
                                  jnp.logical_or(diag_if_zero != 0, tri))
            return (s + jnp.sum(jnp.where(pos, sim_prev, 0.0)),
                    cnt + jnp.sum(jnp.where(pos, 1.0, 0.0)))

        def body_i(i, c2):
            s, cnt, sim_prev = c2
            a = cache_ref[pl.ds(pl.multiple_of(i * block, block), block), :]
            sim = lax.dot_general(
                a, bn, (((1,), (1,)), ((), ())),
                preferred_element_type=jnp.float32,
            )
            s, cnt = reduce_tile(sim_prev, i, (s, cnt))
            return (s, cnt, sim)

        s, cnt, sim_last = lax.fori_loop(0, k, body_i, c1 + (sim_d,))
        return reduce_tile(sim_last, k, (s, cnt))

    return lax.fori_loop(0, nb, body_k, carry)


def _both_layers_kernel(w0_hbm, w1_hbm, out_ref, c0_ref, c1_ref, stg0_ref,
                        stg1_ref, sem0, sem1, *, block, margin, eps):
    _copy_for(w0_hbm, stg0_ref, sem0, block, 0, 0).start()
    _copy_for(w1_hbm, stg1_ref, sem1, block, 0, 0).start()
    zero = (jnp.float32(0.0), jnp.float32(0.0))
    s0, c0 = _stream_layer(w0_hbm, c0_ref, stg0_ref, sem0, block, margin,
                           eps, zero)
    s1, c1 = _stream_layer(w1_hbm, c1_ref, stg1_ref, sem1, block, margin,
                           eps, zero)
    out_ref[...] = jnp.concatenate([
        s0.reshape(1, 1), c0.reshape(1, 1),
        s1.reshape(1, 1), c1.reshape(1, 1)], axis=1)


def kernel(w0, w1):
    block = 256
    n0, d0 = w0.shape
    n1, d1 = w1.shape
    out = pl.pallas_call(
        functools.partial(_both_layers_kernel, block=block, margin=_MARGIN,
                          eps=_EPS),
        in_specs=[
            pl.BlockSpec(memory_space=pltpu.MemorySpace.HBM),
            pl.BlockSpec(memory_space=pltpu.MemorySpace.HBM),
        ],
        out_specs=pl.BlockSpec((1, 4), lambda: (0, 0)),
        out_shape=jax.ShapeDtypeStruct((1, 4), jnp.float32),
        scratch_shapes=[
            pltpu.VMEM((n0, d0), jnp.float8_e4m3fn),
            pltpu.VMEM((n1, d1), jnp.float8_e4m3fn),
            pltpu.VMEM((2, block, d0), jnp.float32),
            pltpu.VMEM((2, block, d1), jnp.float32),
            pltpu.SemaphoreType.DMA((2,)),
            pltpu.SemaphoreType.DMA((2,)),
        ],
    )(w0, w1)
    s0, c0, s1, c1 = out[0, 0], out[0, 1], out[0, 2], out[0, 3]
    hs0 = s0 - _MARGIN * c0
    hs1 = s1 - _MARGIN * c1
    l0 = hs0 / jnp.maximum(c0, 1.0)
    l1 = hs1 / jnp.maximum(c1, 1.0)
    return 0.5 * (l0 + l1)


# speedup vs baseline: 1.2372x; 1.2372x over previous
import functools

import jax
import jax.numpy as jnp
from jax import lax
from jax.experimental import pallas as pl
from jax.experimental.pallas import tpu as pltpu

_MARGIN = 0.02
_EPS = 1e-8


def _copy_for(w_hbm, stg_ref, sem, block, k, slot):
    return pltpu.make_async_copy(
        w_hbm.at[pl.ds(k * block, block), :], stg_ref.at[slot],
        sem.at[slot])


def _stream_layer(w_hbm, cache_ref, stg_ref, sem, block, margin, eps, carry):
    n, d = w_hbm.shape
    nb = n // block
    tile_shape = (block, block)
    tri = (lax.broadcasted_iota(jnp.int32, tile_shape, 1) >
           lax.broadcasted_iota(jnp.int32, tile_shape, 0))

    def body_k(k, c1):
        slot = lax.rem(k, 2)

        @pl.when(k + 1 < nb)
        def _prefetch():
            _copy_for(w_hbm, stg_ref, sem, block, k + 1,
                      lax.rem(k + 1, 2)).start()

        _copy_for(w_hbm, stg_ref, sem, block, k, slot).wait()
        f = stg_ref[slot]
        inv_k = 1.0 / jnp.maximum(jnp.sqrt(jnp.sum(f * f, axis=1)), eps)
        bn = (f * inv_k[:, None]).astype(jnp.float8_e4m3fn)
        cache_ref[pl.ds(pl.multiple_of(k * block, block), block), :] = bn

        sim_d = lax.dot_general(
            bn, bn, (((1,), (1,)), ((), ())),
            preferred_element_type=jnp.float32,
        )

        def reduce_tile(sim_prev, diag_if_zero, c2):
            s, cnt = c2
            pos = jnp.logical_and(sim_prev > margin,
                                  jnp.logical_or(diag_if_zero != 0, tri))
            return (s + jnp.sum(jnp.where(pos, sim_prev, 0.0)),
                    cnt + jnp.sum(jnp.where(pos, 1.0, 0.0)))

        def body_i(i, c2):
            s, cnt, sim_prev = c2
            a = cache_ref[pl.ds(pl.multiple_of(i * block, block), block), :]
            sim = lax.dot_general(
                a, bn, (((1,), (1,)), ((), ())),
                preferred_element_type=jnp.float32,
            )
            s, cnt = reduce_tile(sim_prev, i, (s, cnt))
            return (s, cnt, sim)

        s, cnt, sim_last = lax.fori_loop(0, k, body_i, c1 + (sim_d,))
        return reduce_tile(sim_last, k, (s, cnt))

    return lax.fori_loop(0, nb, body_k, carry)


def _both_layers_kernel(w0_hbm, w1_hbm, out_ref, c0_ref, c1_ref, stg0_ref,
                        stg1_ref, sem0, sem1, *, block, margin, eps):
    _copy_for(w0_hbm, stg0_ref, sem0, block, 0, 0).start()
    _copy_for(w1_hbm, stg1_ref, sem1, block, 0, 0).start()
    zero = (jnp.float32(0.0), jnp.float32(0.0))
    s0, c0 = _stream_layer(w0_hbm, c0_ref, stg0_ref, sem0, block, margin,
                           eps, zero)
    s1, c1 = _stream_layer(w1_hbm, c1_ref, stg1_ref, sem1, block, margin,
                           eps, zero)
    out_ref[...] = jnp.concatenate([
        s0.reshape(1, 1), c0.reshape(1, 1),
        s1.reshape(1, 1), c1.reshape(1, 1)], axis=1)


def kernel(w0, w1):
    block = 512
    n0, d0 = w0.shape
    n1, d1 = w1.shape
    out = pl.pallas_call(
        functools.partial(_both_layers_kernel, block=block, margin=_MARGIN,
                          eps=_EPS),
        in_specs=[
            pl.BlockSpec(memory_space=pltpu.MemorySpace.HBM),
            pl.BlockSpec(memory_space=pltpu.MemorySpace.HBM),
        ],
        out_specs=pl.BlockSpec((1, 4), lambda: (0, 0)),
        out_shape=jax.ShapeDtypeStruct((1, 4), jnp.float32),
        scratch_shapes=[
            pltpu.VMEM((n0, d0), jnp.float8_e4m3fn),
            pltpu.VMEM((n1, d1), jnp.float8_e4m3fn),
            pltpu.VMEM((2, block, d0), jnp.float32),
            pltpu.VMEM((2, block, d1), jnp.float32),
            pltpu.SemaphoreType.DMA((2,)),
            pltpu.SemaphoreType.DMA((2,)),
        ],
    )(w0, w1)
    s0, c0, s1, c1 = out[0, 0], out[0, 1], out[0, 2], out[0, 3]
    hs0 = s0 - _MARGIN * c0
    hs1 = s1 - _MARGIN * c1
    l0 = hs0 / jnp.maximum(c0, 1.0)
    l1 = hs1 / jnp.maximum(c1, 1.0)
    return 0.5 * (l0 + l1)
